# baseline (device time: 40172 ns/iter reference)
import jax
import jax.numpy as jnp
from jax import lax
from jax.experimental import pallas as pl
from jax.experimental.pallas import tpu as pltpu

N_DEV = 4
B = 64
D = 1024
H = 2048
HALF = D // 2
BG = N_DEV * B
N_SLOT = 8
N_SEM = 3 * N_SLOT


def kernel(x, Win0, Wout0, Win1, Wout1, Win2, Wout2):
    def body(x_ref, win0, wout0, win1, wout1, win2, wout2, out_ref,
             xfull, part, sbuf, rbuf, winv, woutv,
             send_sems, recv_sems, wsems):
        my = lax.axis_index("i")

        barrier = pltpu.get_barrier_semaphore()
        for d in (1, 2, 3):
            pl.semaphore_signal(barrier, inc=1, device_id=(my ^ d,),
                                device_id_type=pl.DeviceIdType.MESH)
        pl.semaphore_wait(barrier, 3)

        hbm_w = ((win0, wout0), (win1, wout1), (win2, wout2))

        def start_wcopy(k):
            cin = pltpu.make_async_copy(hbm_w[k][0], winv.at[k % 2],
                                        wsems.at[2 * k])
            cout = pltpu.make_async_copy(hbm_w[k][1], woutv.at[k % 2],
                                         wsems.at[2 * k + 1])
            cin.start()
            cout.start()
            return (cin, cout)

        phase_ctr = [0]

        def make_rdma(ph, d, src, dst):
            import os
            if os.environ.get("SKIP_COMM"):
                class _Noop:
                    def start(self): pass
                    def wait(self): pass
                    def wait_send(self): pass
                    def wait_recv(self): pass
                return _Noop()
            i = 3 * (ph % N_SLOT) + (d - 1)
            return pltpu.make_async_remote_copy(
                src_ref=src, dst_ref=dst,
                send_sem=send_sems.at[i], recv_sem=recv_sems.at[i],
                device_id=(my ^ d,), device_id_type=pl.DeviceIdType.MESH,
            )

        def start_ag(c):
            ph = phase_ctr[0]
            phase_ctr[0] += 1
            rdmas = {}
            for d in (1, 2, 3):
                rdmas[d] = make_rdma(ph, d, xfull.at[c, 0], xfull.at[c, d])
                rdmas[d].start()
            return rdmas

        def start_rs(c):
            ph = phase_ctr[0]
            phase_ctr[0] += 1
            rdmas = {}
            for d in (1, 2, 3):
                sbuf[c, d - 1, :, :] = part[c, d, :, :].astype(jnp.bfloat16)
            for d in (1, 2, 3):
                rdmas[d] = make_rdma(ph, d, sbuf.at[c, d - 1],
                                     rbuf.at[c, d - 1])
                rdmas[d].start()
            return rdmas

        def reduce(c, rs):
            for d in (1, 3, 2):
                rs[d].wait_recv()
            return (part[c, 0, :, :]
                    + rbuf[c, 0, :, :].astype(jnp.float32)
                    + rbuf[c, 1, :, :].astype(jnp.float32)
                    + rbuf[c, 2, :, :].astype(jnp.float32))

        wc0 = start_wcopy(0)
        wc1 = start_wcopy(1)
        xb = x_ref[:, :].astype(jnp.bfloat16)
        xfull[0, 0, :, :] = xb[:, :HALF]
        xfull[1, 0, :, :] = xb[:, HALF:]
        ag = {0: start_ag(0), 1: start_ag(1)}
        wcopies = [wc0, wc1]

        for k in range(3):
            for c in wcopies[k]:
                c.wait()
            winb = winv.at[k % 2]
            woutb = woutv.at[k % 2]
            for d in (1, 3, 2):
                ag[0][d].wait_recv()
            h0 = jnp.dot(
                xfull[0, :, :, :].reshape(BG, HALF).astype(jnp.float32),
                winb[:HALF, :], preferred_element_type=jnp.float32)
            for d in (1, 3, 2):
                ag[1][d].wait_recv()
            h = jnp.maximum(
                h0 + jnp.dot(
                    xfull[1, :, :, :].reshape(BG, HALF).astype(jnp.float32),
                    winb[HALF:, :], preferred_element_type=jnp.float32),
                0.0)
            for d in (1, 2, 3):
                ag[0][d].wait_send()
                ag[1][d].wait_send()
            part[0, :, :, :] = jnp.dot(
                h, woutb[:, :HALF],
                preferred_element_type=jnp.float32).reshape(N_DEV, B, HALF)
            rs0 = start_rs(0)
            part[1, :, :, :] = jnp.dot(
                h, woutb[:, HALF:],
                preferred_element_type=jnp.float32).reshape(N_DEV, B, HALF)
            rs1 = start_rs(1)
            if k == 0:
                wcopies.append(start_wcopy(2))
            red0 = reduce(0, rs0)
            if k < 2:
                xfull[0, 0, :, :] = red0.astype(jnp.bfloat16)
                ag = {0: start_ag(0)}
                red1 = reduce(1, rs1)
                xfull[1, 0, :, :] = red1.astype(jnp.bfloat16)
                ag[1] = start_ag(1)
            else:
                out_ref[:, :HALF] = red0
                red1 = reduce(1, rs1)
                out_ref[:, HALF:] = red1
            for d in (1, 2, 3):
                rs0[d].wait_send()
                rs1[d].wait_send()

    return pl.pallas_call(
        body,
        out_shape=jax.ShapeDtypeStruct((B, D), jnp.float32),
        in_specs=[pl.BlockSpec(memory_space=pltpu.VMEM)]
        + [pl.BlockSpec(memory_space=pltpu.MemorySpace.HBM)] * 6,
        out_specs=pl.BlockSpec(memory_space=pltpu.VMEM),
        scratch_shapes=[
            pltpu.VMEM((2, N_DEV, B, HALF), jnp.bfloat16),
            pltpu.VMEM((2, N_DEV, B, HALF), jnp.float32),
            pltpu.VMEM((2, 3, B, HALF), jnp.bfloat16),
            pltpu.VMEM((2, 3, B, HALF), jnp.bfloat16),
            pltpu.VMEM((2, D, H), jnp.float32),
            pltpu.VMEM((2, H, D), jnp.float32),
            pltpu.SemaphoreType.DMA((N_SEM,)),
            pltpu.SemaphoreType.DMA((N_SEM,)),
            pltpu.SemaphoreType.DMA((6,)),
        ],
        compiler_params=pltpu.CompilerParams(
            collective_id=0,
            vmem_limit_bytes=100 * 1024 * 1024,
        ),
    )(x, Win0, Wout0, Win1, Wout1, Win2, Wout2)


# device time: 39910 ns/iter; 1.0066x vs baseline; 1.0066x over previous
import jax
import jax.numpy as jnp
from jax import lax
from jax.experimental import pallas as pl
from jax.experimental.pallas import tpu as pltpu

N_DEV = 4
B = 64
D = 1024
H = 2048
HALF = D // 2
BG = N_DEV * B
N_SLOT = 8
N_SEM = 3 * N_SLOT


def kernel(x, Win0, Wout0, Win1, Wout1, Win2, Wout2):
    def body(x_ref, win0, wout0, win1, wout1, win2, wout2, out_ref,
             xfull, xcat, part, sbuf, rbuf, winv, woutv,
             send_sems, recv_sems, wsems):
        my = lax.axis_index("i")

        barrier = pltpu.get_barrier_semaphore()
        for d in (1, 2, 3):
            pl.semaphore_signal(barrier, inc=1, device_id=(my ^ d,),
                                device_id_type=pl.DeviceIdType.MESH)
        pl.semaphore_wait(barrier, 3)

        hbm_w = ((win0, wout0), (win1, wout1), (win2, wout2))

        def start_wcopy(k):
            cin = pltpu.make_async_copy(hbm_w[k][0], winv.at[k % 2],
                                        wsems.at[2 * k])
            cout = pltpu.make_async_copy(hbm_w[k][1], woutv.at[k % 2],
                                         wsems.at[2 * k + 1])
            cin.start()
            cout.start()
            return (cin, cout)

        phase_ctr = [0]

        def make_rdma(ph, d, src, dst):
            import os
            if os.environ.get("SKIP_COMM"):
                class _Noop:
                    def start(self): pass
                    def wait(self): pass
                    def wait_send(self): pass
                    def wait_recv(self): pass
                return _Noop()
            i = 3 * (ph % N_SLOT) + (d - 1)
            return pltpu.make_async_remote_copy(
                src_ref=src, dst_ref=dst,
                send_sem=send_sems.at[i], recv_sem=recv_sems.at[i],
                device_id=(my ^ d,), device_id_type=pl.DeviceIdType.MESH,
            )

        def start_ag(c):
            ph = phase_ctr[0]
            phase_ctr[0] += 1
            rdmas = {}
            for d in (1, 2, 3):
                rdmas[d] = make_rdma(ph, d, xfull.at[c, 0], xfull.at[c, d])
                rdmas[d].start()
            return rdmas

        def start_rs(c):
            ph = phase_ctr[0]
            phase_ctr[0] += 1
            rdmas = {}
            for d in (1, 2, 3):
                sbuf[c, d - 1, :, :] = part[c, d, :, :].astype(jnp.bfloat16)
            for d in (1, 2, 3):
                rdmas[d] = make_rdma(ph, d, sbuf.at[c, d - 1],
                                     rbuf.at[c, d - 1])
                rdmas[d].start()
            return rdmas

        def reduce(c, rs):
            for d in (1, 3, 2):
                rs[d].wait_recv()
            return (part[c, 0, :, :]
                    + rbuf[c, 0, :, :].astype(jnp.float32)
                    + rbuf[c, 1, :, :].astype(jnp.float32)
                    + rbuf[c, 2, :, :].astype(jnp.float32))

        wc0 = start_wcopy(0)
        wc1 = start_wcopy(1)
        xb = x_ref[:, :].astype(jnp.bfloat16)
        xfull[0, 0, :, :] = xb[:, :HALF]
        xfull[1, 0, :, :] = xb[:, HALF:]
        ag = {0: start_ag(0), 1: start_ag(1)}
        wcopies = [wc0, wc1]

        for k in range(3):
            for c in wcopies[k]:
                c.wait()
            winb = winv.at[k % 2]
            woutb = woutv.at[k % 2]
            for d in (1, 3, 2):
                ag[0][d].wait_recv()
                ag[1][d].wait_recv()
            xcat[:, :HALF] = xfull[0, :, :, :].reshape(BG, HALF)
            xcat[:, HALF:] = xfull[1, :, :, :].reshape(BG, HALF)
            h = jnp.maximum(
                jnp.dot(xcat[:, :].astype(jnp.float32), winb[:, :],
                        preferred_element_type=jnp.float32), 0.0)
            for d in (1, 2, 3):
                ag[0][d].wait_send()
                ag[1][d].wait_send()
            part[0, :, :, :] = jnp.dot(
                h, woutb[:, :HALF],
                preferred_element_type=jnp.float32).reshape(N_DEV, B, HALF)
            rs0 = start_rs(0)
            part[1, :, :, :] = jnp.dot(
                h, woutb[:, HALF:],
                preferred_element_type=jnp.float32).reshape(N_DEV, B, HALF)
            rs1 = start_rs(1)
            if k == 0:
                wcopies.append(start_wcopy(2))
            red0 = reduce(0, rs0)
            if k < 2:
                xfull[0, 0, :, :] = red0.astype(jnp.bfloat16)
                ag = {0: start_ag(0)}
                red1 = reduce(1, rs1)
                xfull[1, 0, :, :] = red1.astype(jnp.bfloat16)
                ag[1] = start_ag(1)
            else:
                out_ref[:, :HALF] = red0
                red1 = reduce(1, rs1)
                out_ref[:, HALF:] = red1
            for d in (1, 2, 3):
                rs0[d].wait_send()
                rs1[d].wait_send()

    return pl.pallas_call(
        body,
        out_shape=jax.ShapeDtypeStruct((B, D), jnp.float32),
        in_specs=[pl.BlockSpec(memory_space=pltpu.VMEM)]
        + [pl.BlockSpec(memory_space=pltpu.MemorySpace.HBM)] * 6,
        out_specs=pl.BlockSpec(memory_space=pltpu.VMEM),
        scratch_shapes=[
            pltpu.VMEM((2, N_DEV, B, HALF), jnp.bfloat16),
            pltpu.VMEM((BG, D), jnp.bfloat16),
            pltpu.VMEM((2, N_DEV, B, HALF), jnp.float32),
            pltpu.VMEM((2, 3, B, HALF), jnp.bfloat16),
            pltpu.VMEM((2, 3, B, HALF), jnp.bfloat16),
            pltpu.VMEM((2, D, H), jnp.float32),
            pltpu.VMEM((2, H, D), jnp.float32),
            pltpu.SemaphoreType.DMA((N_SEM,)),
            pltpu.SemaphoreType.DMA((N_SEM,)),
            pltpu.SemaphoreType.DMA((6,)),
        ],
        compiler_params=pltpu.CompilerParams(
            collective_id=0,
            vmem_limit_bytes=100 * 1024 * 1024,
        ),
    )(x, Win0, Wout0, Win1, Wout1, Win2, Wout2)
